# baseline (speedup 1.0000x reference)
import functools

import jax
import jax.numpy as jnp
from jax import lax
from jax.experimental import pallas as pl
from jax.experimental.pallas import tpu as pltpu
from jax.experimental.pallas import tpu_sc as plsc

_N = 512
_G = 32
_E = 8192

_INTERP = False

_NW = 32
_EPW = _E // _NW
_CH = _N * _N // 16


def _adj_sc_body(ei_hbm, out_hbm, src_v, dst_v, idx_v, val_v, zero_v, acc_sh):
    c = lax.axis_index("c")
    s = lax.axis_index("s")
    wid = s * 2 + c
    base = wid * _EPW

    pltpu.sync_copy(ei_hbm.at[0, pl.ds(base, _EPW)], src_v)
    pltpu.sync_copy(ei_hbm.at[1, pl.ds(base, _EPW)], dst_v)

    one = jnp.full((16,), 1.0, jnp.float32)
    for i in range(_EPW // 16):
        sl = pl.ds(i * 16, 16)
        idx_v[sl] = dst_v[sl] * _N + src_v[sl]
        val_v[sl] = one
    n = lax.iota(jnp.int32, 16) + wid * 16
    idx_v[pl.ds(_EPW, 16)] = n * (_N + 1)
    val_v[pl.ds(_EPW, 16)] = one

    def zbody(i, carry):
        zero_v[pl.ds(i * 16, 16)] = jnp.zeros((16,), jnp.float32)
        return carry
    lax.fori_loop(0, _CH // 16, zbody, 0)
    pltpu.sync_copy(zero_v, acc_sh.at[pl.ds(s * _CH, _CH)])
    plsc.subcore_barrier()

    pltpu.sync_copy(val_v, acc_sh.at[idx_v], add=True)
    plsc.subcore_barrier()

    pltpu.sync_copy(acc_sh.at[pl.ds(s * _CH, _CH)],
                    out_hbm.at[c, pl.ds(s * _CH, _CH)])


def _build_adj_sc(edge_index):
    mesh = plsc.VectorSubcoreMesh(core_axis_name="c", subcore_axis_name="s")
    k = pl.kernel(
        _adj_sc_body,
        mesh=mesh,
        out_type=jax.ShapeDtypeStruct((2, _N * _N), jnp.float32),
        scratch_types=[
            pltpu.VMEM((_EPW,), jnp.int32),
            pltpu.VMEM((_EPW,), jnp.int32),
            pltpu.VMEM((_EPW + 16,), jnp.int32),
            pltpu.VMEM((_EPW + 16,), jnp.float32),
            pltpu.VMEM((_CH,), jnp.float32),
            pltpu.VMEM_SHARED((_N * _N,), jnp.float32),
        ],
    )
    return k(edge_index)


def _gat_kernel(x_ref, m_ref, w_ref, as_ref, ad_ref, b_ref, o_ref,
                *, heads, out_ch, pool):
    xg = x_ref[0]
    h = jnp.dot(xg, w_ref[...], preferred_element_type=jnp.float32)
    asrc = jnp.dot(h, as_ref[...], preferred_element_type=jnp.float32,
                   precision=jax.lax.Precision.HIGHEST)
    adst = jnp.dot(h, ad_ref[...], preferred_element_type=jnp.float32,
                   precision=jax.lax.Precision.HIGHEST)
    asrc_t = jnp.transpose(asrc)
    mm = m_ref[0] + m_ref[1]
    outs = []
    for k in range(heads):
        logit = adst[:, k:k + 1] + asrc_t[k:k + 1, :]
        alpha = jnp.where(logit >= 0.0, jnp.minimum(logit, 60.0),
                          0.2 * logit)
        e = jnp.exp(alpha) * mm
        rdenom = 1.0 / (jnp.sum(e, axis=1, keepdims=True) + 1e-16)
        hk = h[:, k * out_ch:(k + 1) * out_ch]
        outs.append(
            jnp.dot(e, hk, preferred_element_type=jnp.float32,
                    precision=jax.lax.Precision.HIGHEST) * rdenom)
    out = outs[0] if heads == 1 else jnp.concatenate(outs, axis=1)
    out = jnp.maximum(out + b_ref[...], 0.0)
    if pool:
        o_ref[0] = jnp.sum(out, axis=0, keepdims=True)
    else:
        o_ref[0] = out


def _gat_layer(x, m, w, a_src, a_dst, b, heads, out_ch, pool):
    cin = x.shape[-1]
    cout = heads * out_ch
    eye = jnp.eye(heads, dtype=jnp.float32)
    a_s = (a_src[:, :, None] * eye[:, None, :]).reshape(cout, heads)
    a_d = (a_dst[:, :, None] * eye[:, None, :]).reshape(cout, heads)
    odim = out_ch if heads == 1 else cout
    oshape = (_G, 1, odim) if pool else (_G, _N, odim)
    ospec = (pl.BlockSpec((1, 1, odim), lambda g: (g, 0, 0)) if pool
             else pl.BlockSpec((1, _N, odim), lambda g: (g, 0, 0)))
    out = pl.pallas_call(
        functools.partial(_gat_kernel, heads=heads, out_ch=out_ch, pool=pool),
        grid=(_G,),
        in_specs=[
            pl.BlockSpec((1, _N, cin), lambda g: (g, 0, 0)),
            pl.BlockSpec((2, _N, _N), lambda g: (0, 0, 0)),
            pl.BlockSpec((cin, cout), lambda g: (0, 0)),
            pl.BlockSpec((cout, heads), lambda g: (0, 0)),
            pl.BlockSpec((cout, heads), lambda g: (0, 0)),
            pl.BlockSpec((1, cout), lambda g: (0, 0)),
        ],
        out_specs=ospec,
        out_shape=jax.ShapeDtypeStruct(oshape, jnp.float32),
        interpret=_INTERP,
    )(x, m, w, a_s, a_d, b.reshape(1, cout))
    return out.reshape(_G, odim) if pool else out


def _lstm_kernel(seq_ref, wih_f_ref, whh_f_ref, bf_ref,
                 wih_b_ref, whh_b_ref, bb_ref, wout_ref, bout_ref, o_ref):
    bs = 4
    hdim = 256
    h_f = jnp.zeros((bs, hdim), jnp.float32)
    c_f = jnp.zeros((bs, hdim), jnp.float32)
    h_b = jnp.zeros((bs, hdim), jnp.float32)
    c_b = jnp.zeros((bs, hdim), jnp.float32)

    def cell(xt, h, c, wih, whh, bias):
        g = (jnp.dot(xt, wih, preferred_element_type=jnp.float32)
             + jnp.dot(h, whh, preferred_element_type=jnp.float32) + bias)
        i = jax.nn.sigmoid(g[:, 0:hdim])
        f = jax.nn.sigmoid(g[:, hdim:2 * hdim])
        gg = jnp.tanh(g[:, 2 * hdim:3 * hdim])
        o = jax.nn.sigmoid(g[:, 3 * hdim:4 * hdim])
        c = f * c + i * gg
        h = o * jnp.tanh(c)
        return h, c

    for t in range(8):
        h_f, c_f = cell(seq_ref[t], h_f, c_f,
                        wih_f_ref[...], whh_f_ref[...], bf_ref[...])
        h_b, c_b = cell(seq_ref[7 - t], h_b, c_b,
                        wih_b_ref[...], whh_b_ref[...], bb_ref[...])

    temporal = jnp.concatenate([h_f, h_b], axis=1)
    o_ref[...] = (jnp.dot(temporal, wout_ref[...],
                          preferred_element_type=jnp.float32)
                  + bout_ref[...])


def _lstm_heads(gemb, wih_f, whh_f, b_f, wih_b, whh_b, b_b, wout, bout):
    seq = jnp.transpose(gemb.reshape(4, 8, 64), (1, 0, 2))
    odim = wout.shape[1]
    return pl.pallas_call(
        _lstm_kernel,
        out_shape=jax.ShapeDtypeStruct((4, odim), jnp.float32),
        interpret=_INTERP,
    )(seq, wih_f, whh_f, b_f.reshape(1, -1),
      wih_b, whh_b, b_b.reshape(1, -1), wout, bout.reshape(1, -1))


def kernel(x, edge_index, W1, a_s1, a_d1, b1, W2, a_s2, a_d2, b2,
           W3, a_s3, a_d3, b3, Wih_f, Whh_f, bih_f, bhh_f,
           Wih_b, Whh_b, bih_b, bhh_b, Wmu, bmu, Wlv, blv, Wpi, bpi):
    m = _build_adj_sc(edge_index).reshape(2, _N, _N)
    xt = x.reshape(_G, _N, 128)
    xt = _gat_layer(xt, m, W1, a_s1, a_d1, b1, 4, 64, False)
    xt = _gat_layer(xt, m, W2, a_s2, a_d2, b2, 4, 64, False)
    gemb = _gat_layer(xt, m, W3, a_s3, a_d3, b3, 1, 64, True)

    wout = jnp.concatenate([Wmu, Wlv, Wpi], axis=0).T
    bout = jnp.concatenate([bmu, blv, bpi], axis=0)
    out = _lstm_heads(gemb, Wih_f.T, Whh_f.T, bih_f + bhh_f,
                      Wih_b.T, Whh_b.T, bih_b + bhh_b, wout, bout)
    mu = out[:, :2048].reshape(4, 32, 64)
    logvar = out[:, 2048:4096].reshape(4, 32, 64)
    pi = out[:, 4096:4128]
    return (mu, logvar, pi)


# speedup vs baseline: 1.4259x; 1.4259x over previous
import functools

import jax
import jax.numpy as jnp
from jax import lax
from jax.experimental import pallas as pl
from jax.experimental.pallas import tpu as pltpu
from jax.experimental.pallas import tpu_sc as plsc

_N = 512
_G = 32
_E = 8192

_INTERP = False

_NW = 32
_EPW = _E // _NW
_CH = _N * _N // 16


def _adj_sc_body(ei_hbm, out_hbm, src_v, dst_v, idx_v, val_v, zero_v, acc_sh):
    c = lax.axis_index("c")
    s = lax.axis_index("s")
    wid = s * 2 + c
    base = wid * _EPW

    pltpu.sync_copy(ei_hbm.at[0, pl.ds(base, _EPW)], src_v)
    pltpu.sync_copy(ei_hbm.at[1, pl.ds(base, _EPW)], dst_v)

    one = jnp.full((16,), 1.0, jnp.float32)
    for i in range(_EPW // 16):
        sl = pl.ds(i * 16, 16)
        idx_v[sl] = dst_v[sl] * _N + src_v[sl]
        val_v[sl] = one
    n = lax.iota(jnp.int32, 16) + wid * 16
    idx_v[pl.ds(_EPW, 16)] = n * (_N + 1)
    val_v[pl.ds(_EPW, 16)] = one

    def zbody(i, carry):
        zero_v[pl.ds(i * 16, 16)] = jnp.zeros((16,), jnp.float32)
        return carry
    lax.fori_loop(0, _CH // 16, zbody, 0)
    pltpu.sync_copy(zero_v, acc_sh.at[pl.ds(s * _CH, _CH)])
    plsc.subcore_barrier()

    pltpu.sync_copy(val_v, acc_sh.at[idx_v], add=True)
    plsc.subcore_barrier()

    pltpu.sync_copy(acc_sh.at[pl.ds(s * _CH, _CH)],
                    out_hbm.at[c, pl.ds(s * _CH, _CH)])


def _build_adj_sc(edge_index):
    mesh = plsc.VectorSubcoreMesh(core_axis_name="c", subcore_axis_name="s")
    k = pl.kernel(
        _adj_sc_body,
        mesh=mesh,
        out_type=jax.ShapeDtypeStruct((2, _N * _N), jnp.float32),
        scratch_types=[
            pltpu.VMEM((_EPW,), jnp.int32),
            pltpu.VMEM((_EPW,), jnp.int32),
            pltpu.VMEM((_EPW + 16,), jnp.int32),
            pltpu.VMEM((_EPW + 16,), jnp.float32),
            pltpu.VMEM((_CH,), jnp.float32),
            pltpu.VMEM_SHARED((_N * _N,), jnp.float32),
        ],
    )
    return k(edge_index)


def _gat_kernel(x_ref, m_ref, w_ref, as_ref, ad_ref, b_ref, o_ref,
                *, heads, out_ch, pool):
    xg = x_ref[0]
    h = jnp.dot(xg, w_ref[...], preferred_element_type=jnp.float32)
    asrc = jnp.dot(h, as_ref[...], preferred_element_type=jnp.float32,
                   precision=jax.lax.Precision.HIGHEST)
    adst = jnp.dot(h, ad_ref[...], preferred_element_type=jnp.float32,
                   precision=jax.lax.Precision.HIGHEST)
    asrc_t = jnp.transpose(asrc)
    mm = m_ref[0] + m_ref[1]
    h_hi = h.astype(jnp.bfloat16)
    h_lo = (h - h_hi.astype(jnp.float32)).astype(jnp.bfloat16)
    outs = []
    for k in range(heads):
        logit = adst[:, k:k + 1] + asrc_t[k:k + 1, :]
        alpha = jnp.where(logit >= 0.0, jnp.minimum(logit, 60.0),
                          0.2 * logit)
        e = jnp.exp(alpha) * mm
        rdenom = 1.0 / (jnp.sum(e, axis=1, keepdims=True) + 1e-16)
        e_hi = e.astype(jnp.bfloat16)
        e_lo = (e - e_hi.astype(jnp.float32)).astype(jnp.bfloat16)
        hk_hi = h_hi[:, k * out_ch:(k + 1) * out_ch]
        hk_lo = h_lo[:, k * out_ch:(k + 1) * out_ch]
        agg = (jnp.dot(e_hi, hk_hi, preferred_element_type=jnp.float32)
               + (jnp.dot(e_hi, hk_lo, preferred_element_type=jnp.float32)
                  + jnp.dot(e_lo, hk_hi, preferred_element_type=jnp.float32)))
        outs.append(agg * rdenom)
    out = outs[0] if heads == 1 else jnp.concatenate(outs, axis=1)
    out = jnp.maximum(out + b_ref[...], 0.0)
    if pool:
        o_ref[0] = jnp.sum(out, axis=0, keepdims=True)
    else:
        o_ref[0] = out


def _gat_layer(x, m, w, a_src, a_dst, b, heads, out_ch, pool):
    cin = x.shape[-1]
    cout = heads * out_ch
    eye = jnp.eye(heads, dtype=jnp.float32)
    a_s = (a_src[:, :, None] * eye[:, None, :]).reshape(cout, heads)
    a_d = (a_dst[:, :, None] * eye[:, None, :]).reshape(cout, heads)
    odim = out_ch if heads == 1 else cout
    oshape = (_G, 1, odim) if pool else (_G, _N, odim)
    ospec = (pl.BlockSpec((1, 1, odim), lambda g: (g, 0, 0)) if pool
             else pl.BlockSpec((1, _N, odim), lambda g: (g, 0, 0)))
    out = pl.pallas_call(
        functools.partial(_gat_kernel, heads=heads, out_ch=out_ch, pool=pool),
        grid=(_G,),
        in_specs=[
            pl.BlockSpec((1, _N, cin), lambda g: (g, 0, 0)),
            pl.BlockSpec((2, _N, _N), lambda g: (0, 0, 0)),
            pl.BlockSpec((cin, cout), lambda g: (0, 0)),
            pl.BlockSpec((cout, heads), lambda g: (0, 0)),
            pl.BlockSpec((cout, heads), lambda g: (0, 0)),
            pl.BlockSpec((1, cout), lambda g: (0, 0)),
        ],
        out_specs=ospec,
        out_shape=jax.ShapeDtypeStruct(oshape, jnp.float32),
        interpret=_INTERP,
    )(x, m, w, a_s, a_d, b.reshape(1, cout))
    return out.reshape(_G, odim) if pool else out


_DNT = (((1,), (1,)), ((), ()))


def _dot_t(a, b):
    return jax.lax.dot_general(a, b, _DNT, preferred_element_type=jnp.float32)


def _lstm_kernel(seq_ref, wih_f_ref, whh_f_ref, bf_ref,
                 wih_b_ref, whh_b_ref, bb_ref,
                 wmu_ref, bmu_ref, wlv_ref, blv_ref, wpi_ref, bpi_ref, o_ref):
    bs = 4
    hdim = 256
    h_f = jnp.zeros((bs, hdim), jnp.float32)
    c_f = jnp.zeros((bs, hdim), jnp.float32)
    h_b = jnp.zeros((bs, hdim), jnp.float32)
    c_b = jnp.zeros((bs, hdim), jnp.float32)

    def cell(xt, h, c, wih, whh, bias):
        g = _dot_t(xt, wih) + _dot_t(h, whh) + bias
        i = jax.nn.sigmoid(g[:, 0:hdim])
        f = jax.nn.sigmoid(g[:, hdim:2 * hdim])
        gg = jnp.tanh(g[:, 2 * hdim:3 * hdim])
        o = jax.nn.sigmoid(g[:, 3 * hdim:4 * hdim])
        c = f * c + i * gg
        h = o * jnp.tanh(c)
        return h, c

    for t in range(8):
        h_f, c_f = cell(seq_ref[t], h_f, c_f,
                        wih_f_ref[...], whh_f_ref[...], bf_ref[...])
        h_b, c_b = cell(seq_ref[7 - t], h_b, c_b,
                        wih_b_ref[...], whh_b_ref[...], bb_ref[...])

    temporal = jnp.concatenate([h_f, h_b], axis=1)
    o_ref[:, 0:2048] = _dot_t(temporal, wmu_ref[...]) + bmu_ref[...]
    o_ref[:, 2048:4096] = _dot_t(temporal, wlv_ref[...]) + blv_ref[...]
    o_ref[:, 4096:4128] = _dot_t(temporal, wpi_ref[...]) + bpi_ref[...]


def _lstm_heads(gemb, wih_f, whh_f, b_f, wih_b, whh_b, b_b,
                wmu, bmu, wlv, blv, wpi, bpi):
    seq = jnp.transpose(gemb.reshape(4, 8, 64), (1, 0, 2))
    return pl.pallas_call(
        _lstm_kernel,
        out_shape=jax.ShapeDtypeStruct((4, 4128), jnp.float32),
        interpret=_INTERP,
    )(seq, wih_f, whh_f, b_f.reshape(1, -1),
      wih_b, whh_b, b_b.reshape(1, -1),
      wmu, bmu.reshape(1, -1), wlv, blv.reshape(1, -1),
      wpi, bpi.reshape(1, -1))


def kernel(x, edge_index, W1, a_s1, a_d1, b1, W2, a_s2, a_d2, b2,
           W3, a_s3, a_d3, b3, Wih_f, Whh_f, bih_f, bhh_f,
           Wih_b, Whh_b, bih_b, bhh_b, Wmu, bmu, Wlv, blv, Wpi, bpi):
    m = _build_adj_sc(edge_index).reshape(2, _N, _N)
    xt = x.reshape(_G, _N, 128)
    xt = _gat_layer(xt, m, W1, a_s1, a_d1, b1, 4, 64, False)
    xt = _gat_layer(xt, m, W2, a_s2, a_d2, b2, 4, 64, False)
    gemb = _gat_layer(xt, m, W3, a_s3, a_d3, b3, 1, 64, True)

    out = _lstm_heads(gemb, Wih_f, Whh_f, bih_f + bhh_f,
                      Wih_b, Whh_b, bih_b + bhh_b,
                      Wmu, bmu, Wlv, blv, Wpi, bpi)
    mu = out[:, :2048].reshape(4, 32, 64)
    logvar = out[:, 2048:4096].reshape(4, 32, 64)
    pi = out[:, 4096:4128]
    return (mu, logvar, pi)


# speedup vs baseline: 1.4455x; 1.0138x over previous
import functools

import jax
import jax.numpy as jnp
from jax import lax
from jax.experimental import pallas as pl
from jax.experimental.pallas import tpu as pltpu
from jax.experimental.pallas import tpu_sc as plsc

_N = 512
_G = 32
_E = 8192

_INTERP = False

_NW = 32
_EPW = _E // _NW
_CH = _N * _N // 16


def _adj_sc_body(ei_hbm, out_hbm, src_v, dst_v, idx_v, val_v, zero_v, acc_sh):
    c = lax.axis_index("c")
    s = lax.axis_index("s")
    wid = s * 2 + c
    base = wid * _EPW

    pltpu.sync_copy(ei_hbm.at[0, pl.ds(base, _EPW)], src_v)
    pltpu.sync_copy(ei_hbm.at[1, pl.ds(base, _EPW)], dst_v)

    one = jnp.full((16,), 1.0, jnp.float32)
    for i in range(_EPW // 16):
        sl = pl.ds(i * 16, 16)
        idx_v[sl] = dst_v[sl] * _N + src_v[sl]
        val_v[sl] = one
    n = lax.iota(jnp.int32, 16) + wid * 16
    idx_v[pl.ds(_EPW, 16)] = n * (_N + 1)
    val_v[pl.ds(_EPW, 16)] = one

    zv = jnp.zeros((16,), jnp.float32)

    def zbody(i, carry):
        for j in range(8):
            zero_v[pl.ds(i * 128 + j * 16, 16)] = zv
        return carry
    lax.fori_loop(0, _CH // 128, zbody, 0)
    pltpu.sync_copy(zero_v, acc_sh.at[pl.ds(s * _CH, _CH)])
    plsc.subcore_barrier()

    pltpu.sync_copy(val_v, acc_sh.at[idx_v], add=True)
    plsc.subcore_barrier()

    pltpu.sync_copy(acc_sh.at[pl.ds(s * _CH, _CH)],
                    out_hbm.at[c, pl.ds(s * _CH, _CH)])


def _sum2_kernel(m2_ref, o_ref):
    o_ref[...] = m2_ref[0] + m2_ref[1]


def _sum_partials(m2):
    return pl.pallas_call(
        _sum2_kernel,
        out_shape=jax.ShapeDtypeStruct((_N, _N), jnp.float32),
        interpret=_INTERP,
    )(m2)


def _build_adj_sc(edge_index):
    mesh = plsc.VectorSubcoreMesh(core_axis_name="c", subcore_axis_name="s")
    k = pl.kernel(
        _adj_sc_body,
        mesh=mesh,
        out_type=jax.ShapeDtypeStruct((2, _N * _N), jnp.float32),
        scratch_types=[
            pltpu.VMEM((_EPW,), jnp.int32),
            pltpu.VMEM((_EPW,), jnp.int32),
            pltpu.VMEM((_EPW + 16,), jnp.int32),
            pltpu.VMEM((_EPW + 16,), jnp.float32),
            pltpu.VMEM((_CH,), jnp.float32),
            pltpu.VMEM_SHARED((_N * _N,), jnp.float32),
        ],
    )
    return k(edge_index)


def _gat_kernel(x_ref, m_ref, w_ref, as_ref, ad_ref, b_ref, o_ref,
                *, heads, out_ch, pool):
    xg = x_ref[0]
    h = jnp.dot(xg, w_ref[...], preferred_element_type=jnp.float32)
    asrc = jnp.dot(h, as_ref[...], preferred_element_type=jnp.float32,
                   precision=jax.lax.Precision.HIGHEST)
    adst = jnp.dot(h, ad_ref[...], preferred_element_type=jnp.float32,
                   precision=jax.lax.Precision.HIGHEST)
    asrc_t = jnp.transpose(asrc)
    mm = m_ref[...]
    h_hi = h.astype(jnp.bfloat16)
    h_lo = (h - h_hi.astype(jnp.float32)).astype(jnp.bfloat16)
    outs = []
    for k in range(heads):
        logit = adst[:, k:k + 1] + asrc_t[k:k + 1, :]
        alpha = jnp.where(logit >= 0.0, jnp.minimum(logit, 60.0),
                          0.2 * logit)
        e = jnp.exp(alpha) * mm
        rdenom = 1.0 / (jnp.sum(e, axis=1, keepdims=True) + 1e-16)
        e_hi = e.astype(jnp.bfloat16)
        e_lo = (e - e_hi.astype(jnp.float32)).astype(jnp.bfloat16)
        hk_hi = h_hi[:, k * out_ch:(k + 1) * out_ch]
        hk_lo = h_lo[:, k * out_ch:(k + 1) * out_ch]
        agg = (jnp.dot(e_hi, hk_hi, preferred_element_type=jnp.float32)
               + (jnp.dot(e_hi, hk_lo, preferred_element_type=jnp.float32)
                  + jnp.dot(e_lo, hk_hi, preferred_element_type=jnp.float32)))
        outs.append(agg * rdenom)
    out = outs[0] if heads == 1 else jnp.concatenate(outs, axis=1)
    out = jnp.maximum(out + b_ref[...], 0.0)
    if pool:
        o_ref[0] = jnp.sum(out, axis=0, keepdims=True)
    else:
        o_ref[0] = out


def _gat_layer(x, m, w, a_src, a_dst, b, heads, out_ch, pool):
    cin = x.shape[-1]
    cout = heads * out_ch
    eye = jnp.eye(heads, dtype=jnp.float32)
    a_s = (a_src[:, :, None] * eye[:, None, :]).reshape(cout, heads)
    a_d = (a_dst[:, :, None] * eye[:, None, :]).reshape(cout, heads)
    odim = out_ch if heads == 1 else cout
    oshape = (_G, 1, odim) if pool else (_G, _N, odim)
    ospec = (pl.BlockSpec((1, 1, odim), lambda g: (g, 0, 0)) if pool
             else pl.BlockSpec((1, _N, odim), lambda g: (g, 0, 0)))
    out = pl.pallas_call(
        functools.partial(_gat_kernel, heads=heads, out_ch=out_ch, pool=pool),
        grid=(_G,),
        in_specs=[
            pl.BlockSpec((1, _N, cin), lambda g: (g, 0, 0)),
            pl.BlockSpec((_N, _N), lambda g: (0, 0)),
            pl.BlockSpec((cin, cout), lambda g: (0, 0)),
            pl.BlockSpec((cout, heads), lambda g: (0, 0)),
            pl.BlockSpec((cout, heads), lambda g: (0, 0)),
            pl.BlockSpec((1, cout), lambda g: (0, 0)),
        ],
        out_specs=ospec,
        out_shape=jax.ShapeDtypeStruct(oshape, jnp.float32),
        interpret=_INTERP,
    )(x, m, w, a_s, a_d, b.reshape(1, cout))
    return out.reshape(_G, odim) if pool else out


_DNT = (((1,), (1,)), ((), ()))


def _dot_t(a, b):
    return jax.lax.dot_general(a, b, _DNT, preferred_element_type=jnp.float32)


def _lstm_kernel(seq_ref, wih_f_ref, whh_f_ref, bf_ref,
                 wih_b_ref, whh_b_ref, bb_ref,
                 wmu_ref, bmu_ref, wlv_ref, blv_ref, wpi_ref, bpi_ref, o_ref):
    bs = 4
    hdim = 256
    h_f = jnp.zeros((bs, hdim), jnp.float32)
    c_f = jnp.zeros((bs, hdim), jnp.float32)
    h_b = jnp.zeros((bs, hdim), jnp.float32)
    c_b = jnp.zeros((bs, hdim), jnp.float32)
    wih_f = jnp.transpose(wih_f_ref[...])
    whh_f = jnp.transpose(whh_f_ref[...])
    wih_b = jnp.transpose(wih_b_ref[...])
    whh_b = jnp.transpose(whh_b_ref[...])

    def cell(xt, h, c, wih, whh, bias):
        g = (jnp.dot(xt, wih, preferred_element_type=jnp.float32)
             + jnp.dot(h, whh, preferred_element_type=jnp.float32) + bias)
        i = jax.nn.sigmoid(g[:, 0:hdim])
        f = jax.nn.sigmoid(g[:, hdim:2 * hdim])
        gg = jnp.tanh(g[:, 2 * hdim:3 * hdim])
        o = jax.nn.sigmoid(g[:, 3 * hdim:4 * hdim])
        c = f * c + i * gg
        h = o * jnp.tanh(c)
        return h, c

    for t in range(8):
        h_f, c_f = cell(seq_ref[t], h_f, c_f, wih_f, whh_f, bf_ref[...])
        h_b, c_b = cell(seq_ref[7 - t], h_b, c_b, wih_b, whh_b, bb_ref[...])

    temporal = jnp.concatenate([h_f, h_b], axis=1)
    o_ref[:, 0:2048] = _dot_t(temporal, wmu_ref[...]) + bmu_ref[...]
    o_ref[:, 2048:4096] = _dot_t(temporal, wlv_ref[...]) + blv_ref[...]
    o_ref[:, 4096:4128] = _dot_t(temporal, wpi_ref[...]) + bpi_ref[...]


def _lstm_heads(gemb, wih_f, whh_f, b_f, wih_b, whh_b, b_b,
                wmu, bmu, wlv, blv, wpi, bpi):
    seq = jnp.transpose(gemb.reshape(4, 8, 64), (1, 0, 2))
    return pl.pallas_call(
        _lstm_kernel,
        out_shape=jax.ShapeDtypeStruct((4, 4128), jnp.float32),
        interpret=_INTERP,
    )(seq, wih_f, whh_f, b_f.reshape(1, -1),
      wih_b, whh_b, b_b.reshape(1, -1),
      wmu, bmu.reshape(1, -1), wlv, blv.reshape(1, -1),
      wpi, bpi.reshape(1, -1))


def kernel(x, edge_index, W1, a_s1, a_d1, b1, W2, a_s2, a_d2, b2,
           W3, a_s3, a_d3, b3, Wih_f, Whh_f, bih_f, bhh_f,
           Wih_b, Whh_b, bih_b, bhh_b, Wmu, bmu, Wlv, blv, Wpi, bpi):
    m = _sum_partials(_build_adj_sc(edge_index).reshape(2, _N, _N))
    xt = x.reshape(_G, _N, 128)
    xt = _gat_layer(xt, m, W1, a_s1, a_d1, b1, 4, 64, False)
    xt = _gat_layer(xt, m, W2, a_s2, a_d2, b2, 4, 64, False)
    gemb = _gat_layer(xt, m, W3, a_s3, a_d3, b3, 1, 64, True)

    out = _lstm_heads(gemb, Wih_f, Whh_f, bih_f + bhh_f,
                      Wih_b, Whh_b, bih_b + bhh_b,
                      Wmu, bmu, Wlv, blv, Wpi, bpi)
    mu = out[:, :2048].reshape(4, 32, 64)
    logvar = out[:, 2048:4096].reshape(4, 32, 64)
    pi = out[:, 4096:4128]
    return (mu, logvar, pi)
